# baseline (speedup 1.0000x reference)
import functools

import jax
import jax.numpy as jnp
from jax.experimental import pallas as pl

_F32 = jnp.float32
_BF16 = jnp.bfloat16


def _conv_body(x_ref, w_ref, s_ref, b_ref, o_ref, *, taps, w_row, m_rows,
               leaky, out_dtype):
    acc = None
    for dy in range(taps):
        xs = x_ref[0, pl.ds(dy * w_row, m_rows), :]
        t = jnp.dot(xs, w_ref[dy], preferred_element_type=_F32)
        acc = t if acc is None else acc + t
    y = acc * s_ref[...] + b_ref[...]
    if leaky:
        y = jnp.where(y > 0, y, _F32(0.1) * y)
    o_ref[0] = y.astype(out_dtype)


def _conv(x3d, wtap, scale, bias, *, w_row, h_out, leaky, out_dtype=_BF16):
    g_num, r_rows, k_dim = x3d.shape
    taps, _, c_out = wtap.shape
    m_rows = h_out * w_row
    assert r_rows == (h_out + taps - 1) * w_row, (x3d.shape, wtap.shape, w_row, h_out)
    body = functools.partial(_conv_body, taps=taps, w_row=w_row,
                             m_rows=m_rows, leaky=leaky, out_dtype=out_dtype)
    return pl.pallas_call(
        body,
        grid=(g_num,),
        in_specs=[
            pl.BlockSpec((1, r_rows, k_dim), lambda g: (g, 0, 0)),
            pl.BlockSpec((taps, k_dim, c_out), lambda g: (0, 0, 0)),
            pl.BlockSpec((1, c_out), lambda g: (0, 0)),
            pl.BlockSpec((1, c_out), lambda g: (0, 0)),
        ],
        out_specs=pl.BlockSpec((1, m_rows, c_out), lambda g: (g, 0, 0)),
        out_shape=jax.ShapeDtypeStruct((g_num, m_rows, c_out), out_dtype),
    )(x3d.astype(_BF16), wtap.astype(_BF16),
      scale.reshape(1, c_out).astype(_F32), bias.reshape(1, c_out).astype(_F32))


def _pool_body(x_ref, o5_ref, o9_ref, o13_ref):
    a = x_ref[0]

    def pool5(v):
        h2, w2 = v.shape[0] - 4, v.shape[1] - 4
        r = v[0:h2]
        for i in range(1, 5):
            r = jnp.maximum(r, v[i:i + h2])
        c = r[:, 0:w2]
        for i in range(1, 5):
            c = jnp.maximum(c, r[:, i:i + w2])
        return c

    m5 = pool5(a)
    m9 = pool5(m5)
    m13 = pool5(m9)
    o5_ref[0] = m5[4:17, 4:17]
    o9_ref[0] = m9[2:15, 2:15]
    o13_ref[0] = m13


def _spp_pools(s):
    n, h, w, c = s.shape
    pad = jnp.full((n, h + 12, w + 12, c), _BF16(-1e30))
    canvas = pad.at[:, 6:6 + h, 6:6 + w, :].set(s)
    shp = jax.ShapeDtypeStruct((n, h, w, c), _BF16)
    return pl.pallas_call(
        _pool_body,
        grid=(n,),
        in_specs=[pl.BlockSpec((1, h + 12, w + 12, c), lambda g: (g, 0, 0, 0))],
        out_specs=[pl.BlockSpec((1, h, w, c), lambda g: (g, 0, 0, 0))] * 3,
        out_shape=[shp, shp, shp],
    )(canvas)


def _fold_s1(x):
    n, h, w, c = x.shape
    xp = jnp.pad(x, ((0, 0), (1, 1), (1, 1), (0, 0)))
    cat = jnp.concatenate([xp[:, :, 0:w], xp[:, :, 1:w + 1], xp[:, :, 2:w + 2]],
                          axis=-1)
    return cat.reshape(n, (h + 2) * w, 3 * c)


def _w_s1(p):
    return jnp.transpose(p, (2, 3, 1, 0)).reshape(3, -1, p.shape[0])


def _fold_s2(x):
    n, h, w, c = x.shape
    hh, ww = h // 2, w // 2
    xp = jnp.pad(x, ((0, 0), (0, 2), (0, 2), (0, 0)))
    v = xp.reshape(n, hh + 1, 2, ww + 1, 2 * c)
    cat = jnp.concatenate(
        [v[:, :, ry, qx:qx + ww, :] for ry in (0, 1) for qx in (0, 1)],
        axis=-1)
    return cat.reshape(n, (hh + 1) * ww, 8 * c)


def _w_s2(p):
    o, c = p.shape[0], p.shape[1]
    wp = jnp.pad(p, ((0, 0), (0, 0), (0, 1), (0, 1)))
    wr = wp.reshape(o, c, 2, 2, 2, 2)
    return jnp.transpose(wr, (2, 3, 4, 5, 1, 0)).reshape(2, 8 * c, o)


def _upsample2(x):
    n, h, w, c = x.shape
    x = jnp.broadcast_to(x[:, :, None, :, None, :], (n, h, 2, w, 2, c))
    return x.reshape(n, 2 * h, 2 * w, c)


def _ones_bias(p):
    return p["scale"], p["bias"]


def _cbl_s2(x, p, *, h_out, w_out):
    return _conv(_fold_s2(x), _w_s2(p["w"]), p["scale"], p["bias"],
                 w_row=w_out, h_out=h_out, leaky=True).reshape(
                     x.shape[0], h_out, w_out, -1)


def _cbl_s1_3x3(x, p):
    n, h, w, _ = x.shape
    return _conv(_fold_s1(x), _w_s1(p["w"]), p["scale"], p["bias"],
                 w_row=w, h_out=h, leaky=True).reshape(n, h, w, -1)


def _pw(x, w_oi11, scale, bias, *, leaky, out_dtype=_BF16):
    n, h, w, c = x.shape
    wt = w_oi11[:, :, 0, 0].T.reshape(1, c, -1)
    out = _conv(x.reshape(n, h * w, c), wt, scale, bias,
                w_row=h * w, h_out=1, leaky=leaky, out_dtype=out_dtype)
    return out.reshape(n, h, w, -1)


def _cbl_1x1(x, p):
    return _pw(x, p["w"], p["scale"], p["bias"], leaky=True)


def kernel(x, params):
    p = params
    n = x.shape[0]
    xh = jnp.transpose(x, (0, 2, 3, 1)).astype(_BF16)

    xp = jnp.pad(xh, ((0, 0), (0, 2), (0, 2), (0, 0)))
    v = xp.reshape(n, 209, 2, 209, 6)
    xb1 = jnp.concatenate(
        [v[:, qy:qy + 208, ry, qx:qx + 208, :]
         for qy, ry in ((0, 0), (0, 1), (1, 0)) for qx in (0, 1)],
        axis=-1).reshape(n * 8, 208 * 208 // 8, 36)
    wp1 = jnp.pad(p["b1"]["w"], ((0, 0), (0, 0), (0, 1), (0, 1)))
    wr1 = wp1.reshape(32, 3, 2, 2, 2, 2)
    wb1 = jnp.stack(
        [jnp.transpose(wr1[:, :, qy, ry, qx, :], (2, 1, 0))
         for qy, ry in ((0, 0), (0, 1), (1, 0)) for qx in (0, 1)],
        axis=0).reshape(1, 36, 32)
    f1 = _conv(xb1, wb1, p["b1"]["scale"], p["b1"]["bias"],
               w_row=208 * 208 // 8, h_out=1, leaky=True)
    f1 = f1.reshape(n, 208, 208, 32)

    f2 = _cbl_s2(f1, p["b2"], h_out=104, w_out=104)
    feat_small = _cbl_s2(f2, p["b3"], h_out=52, w_out=52)
    feat_medium = _cbl_s2(feat_small, p["b4"], h_out=26, w_out=26)
    feat_large = _cbl_s2(feat_medium, p["b5"], h_out=13, w_out=13)

    s = _cbl_1x1(feat_large, p["spp_c1"])
    m5, m9, m13 = _spp_pools(s)
    s_cat = jnp.concatenate([s, m5, m9, m13], axis=-1)
    p5 = _cbl_1x1(s_cat, p["spp_c2"])

    p5_up = _upsample2(_cbl_1x1(p5, p["conv_up1"]))
    p4 = _cbl_1x1(feat_medium, p["lateral1"])
    p4 = jnp.concatenate([p4, p5_up], axis=-1)
    p4 = _cbl_1x1(p4, p["merge1_0"])
    p4 = _cbl_s1_3x3(p4, p["merge1_1"])
    p4 = _cbl_1x1(p4, p["merge1_2"])

    p4_up = _upsample2(_cbl_1x1(p4, p["conv_up2"]))
    p3 = _cbl_1x1(feat_small, p["lateral2"])
    p3 = jnp.concatenate([p3, p4_up], axis=-1)
    p3 = _cbl_1x1(p3, p["merge2_0"])
    p3 = _cbl_s1_3x3(p3, p["merge2_1"])
    p3 = _cbl_1x1(p3, p["merge2_2"])

    def head(feat, p0, p1):
        h = _cbl_s1_3x3(feat, p0)
        c_out = p1["w"].shape[0]
        out = _pw(h, p1["w"], jnp.ones((c_out,), _F32), p1["b"], leaky=False,
                  out_dtype=_F32)
        return jnp.transpose(out, (0, 3, 1, 2))

    out_small = head(p3, p["head_s_0"], p["head_s_1"])
    out_medium = head(p4, p["head_m_0"], p["head_m_1"])
    out_large = head(p5, p["head_l_0"], p["head_l_1"])
    return (out_small, out_medium, out_large)


# speedup vs baseline: 2.3595x; 2.3595x over previous
import functools

import jax
import jax.numpy as jnp
from jax.experimental import pallas as pl

_F32 = jnp.float32
_BF16 = jnp.bfloat16


def _epilogue(acc, s_ref, b_ref, leaky, out_dtype):
    y = acc * s_ref[...] + b_ref[...]
    if leaky:
        y = jnp.where(y > 0, y, _F32(0.1) * y)
    return y.astype(out_dtype)


def _call_conv(body, xs, wtap, scale, bias, m_rows, out_dtype):
    g_num = xs[0].shape[0]
    k_dim, c_out = wtap.shape
    return pl.pallas_call(
        body,
        grid=(g_num,),
        in_specs=[pl.BlockSpec((1,) + x.shape[1:], lambda g: (g, 0, 0))
                  for x in xs] + [
            pl.BlockSpec((k_dim, c_out), lambda g: (0, 0)),
            pl.BlockSpec((1, c_out), lambda g: (0, 0)),
            pl.BlockSpec((1, c_out), lambda g: (0, 0)),
        ],
        out_specs=pl.BlockSpec((1, m_rows, c_out), lambda g: (g, 0, 0)),
        out_shape=jax.ShapeDtypeStruct((g_num, m_rows, c_out), out_dtype),
    )(*[x.astype(_BF16) for x in xs], wtap.astype(_BF16),
      scale.reshape(1, c_out).astype(_F32), bias.reshape(1, c_out).astype(_F32))


def _pw_body(*refs, n_in, leaky, out_dtype):
    x_refs, w_ref, s_ref, b_ref, o_ref = refs[:n_in], refs[n_in], refs[n_in + 1], refs[n_in + 2], refs[n_in + 3]
    if n_in == 1:
        x = x_refs[0][0]
    else:
        x = jnp.concatenate([r[0] for r in x_refs], axis=-1)
    acc = jnp.dot(x, w_ref[...], preferred_element_type=_F32)
    o_ref[0] = _epilogue(acc, s_ref, b_ref, leaky, out_dtype)


def _pw(xs, w_oi11, scale, bias, *, leaky, out_dtype=_BF16):
    n, h, w, _ = xs[0].shape
    c_out, c_in = w_oi11.shape[0], w_oi11.shape[1]
    wt = w_oi11[:, :, 0, 0].T
    body = functools.partial(_pw_body, n_in=len(xs), leaky=leaky,
                             out_dtype=out_dtype)
    out = _call_conv(body, [x.reshape(n, h * w, x.shape[3]) for x in xs],
                     wt, scale, bias, h * w, out_dtype)
    return out.reshape(n, h, w, c_out)


def _s1_body(x_ref, w_ref, s_ref, b_ref, o_ref, *, h, w, c, leaky, out_dtype):
    m = h * w
    a = x_ref[0]
    z = jnp.zeros((w + 1, c), _BF16)
    ax = jnp.concatenate([z, a, z], axis=0)
    col = jax.lax.broadcasted_iota(jnp.int32, (m, 1), 0) % w
    first, last = col == 0, col == (w - 1)
    parts = []
    for dy in range(3):
        for dx in range(3):
            s = jax.lax.slice(ax, (dy * w + dx, 0), (dy * w + dx + m, c))
            if dx == 0:
                s = jnp.where(first, _BF16(0), s)
            elif dx == 2:
                s = jnp.where(last, _BF16(0), s)
            parts.append(s)
    x = jnp.concatenate(parts, axis=-1)
    acc = jnp.dot(x, w_ref[...], preferred_element_type=_F32)
    o_ref[0] = _epilogue(acc, s_ref, b_ref, leaky, out_dtype)


def _cbl_s1_3x3(x, p):
    n, h, w, c = x.shape
    wt = jnp.transpose(p["w"], (2, 3, 1, 0)).reshape(9 * c, -1)
    body = functools.partial(_s1_body, h=h, w=w, c=c, leaky=True,
                             out_dtype=_BF16)
    out = _call_conv(body, [x.reshape(n, h * w, c)], wt, p["scale"], p["bias"],
                     h * w, _BF16)
    return out.reshape(n, h, w, -1)


def _s2_body(x_ref, w_ref, s_ref, b_ref, o_ref, *, h, w, c, leaky, out_dtype):
    h2, w2 = h // 2, w // 2
    m = h2 * w2
    a = x_ref[0].reshape(h2, 2, w2, 2, c)
    planes = [a[:, ry, :, rx, :].reshape(m, c)
              for ry in (0, 1) for rx in (0, 1)]
    p4 = jnp.concatenate(planes, axis=-1)
    ext = jnp.concatenate([p4, jnp.zeros((w2 + 1, 4 * c), _BF16)], axis=0)
    col = jax.lax.broadcasted_iota(jnp.int32, (m, 1), 0) % w2
    last = col == (w2 - 1)
    parts = []
    for qy in (0, 1):
        for qx in (0, 1):
            s = jax.lax.slice(ext, (qy * w2 + qx, 0), (qy * w2 + qx + m, 4 * c))
            if qx == 1:
                s = jnp.where(last, _BF16(0), s)
            parts.append(s)
    x = jnp.concatenate(parts, axis=-1)
    acc = jnp.dot(x, w_ref[...], preferred_element_type=_F32)
    o_ref[0] = _epilogue(acc, s_ref, b_ref, leaky, out_dtype)


def _cbl_s2(x, p):
    n, h, w, c = x.shape
    o = p["w"].shape[0]
    wp = jnp.pad(p["w"], ((0, 0), (0, 0), (0, 1), (0, 1)))
    wr = wp.reshape(o, c, 2, 2, 2, 2)
    wt = jnp.transpose(wr, (2, 4, 3, 5, 1, 0)).reshape(16 * c, o)
    body = functools.partial(_s2_body, h=h, w=w, c=c, leaky=True,
                             out_dtype=_BF16)
    out = _call_conv(body, [x.reshape(n, h * w, c)], wt, p["scale"], p["bias"],
                     (h // 2) * (w // 2), _BF16)
    return out.reshape(n, h // 2, w // 2, o)


def _pool_body(x_ref, o5_ref, o9_ref, o13_ref):
    a = x_ref[0]

    def pool5(v):
        h2, w2 = v.shape[0] - 4, v.shape[1] - 4
        r = v[0:h2]
        for i in range(1, 5):
            r = jnp.maximum(r, v[i:i + h2])
        cc = r[:, 0:w2]
        for i in range(1, 5):
            cc = jnp.maximum(cc, r[:, i:i + w2])
        return cc

    m5 = pool5(a)
    m9 = pool5(m5)
    m13 = pool5(m9)
    o5_ref[0] = m5[4:17, 4:17]
    o9_ref[0] = m9[2:15, 2:15]
    o13_ref[0] = m13


def _spp_pools(s):
    n, h, w, c = s.shape
    pad = jnp.full((n, h + 12, w + 12, c), _BF16(-1e30))
    canvas = pad.at[:, 6:6 + h, 6:6 + w, :].set(s)
    shp = jax.ShapeDtypeStruct((n, h, w, c), _BF16)
    return pl.pallas_call(
        _pool_body,
        grid=(n,),
        in_specs=[pl.BlockSpec((1, h + 12, w + 12, c), lambda g: (g, 0, 0, 0))],
        out_specs=[pl.BlockSpec((1, h, w, c), lambda g: (g, 0, 0, 0))] * 3,
        out_shape=[shp, shp, shp],
    )(canvas)


def _upsample2(x):
    n, h, w, c = x.shape
    x = jnp.broadcast_to(x[:, :, None, :, None, :], (n, h, 2, w, 2, c))
    return x.reshape(n, 2 * h, 2 * w, c)


def kernel(x, params):
    p = params
    n = x.shape[0]
    xh = jnp.transpose(x, (0, 2, 3, 1)).astype(_BF16)

    xp = jnp.pad(xh, ((0, 0), (0, 2), (0, 2), (0, 0)))
    v = xp.reshape(n, 209, 2, 209, 6)
    xb1 = jnp.concatenate(
        [v[:, qy:qy + 208, ry, qx:qx + 208, :]
         for qy, ry in ((0, 0), (0, 1), (1, 0)) for qx in (0, 1)],
        axis=-1).reshape(n * 8, 208 * 208 // 8, 36)
    wp1 = jnp.pad(p["b1"]["w"], ((0, 0), (0, 0), (0, 1), (0, 1)))
    wr1 = wp1.reshape(32, 3, 2, 2, 2, 2)
    wb1 = jnp.stack(
        [jnp.transpose(wr1[:, :, qy, ry, qx, :], (2, 1, 0))
         for qy, ry in ((0, 0), (0, 1), (1, 0)) for qx in (0, 1)],
        axis=0).reshape(36, 32)
    body1 = functools.partial(_pw_body, n_in=1, leaky=True, out_dtype=_BF16)
    f1 = _call_conv(body1, [xb1], wb1, p["b1"]["scale"], p["b1"]["bias"],
                    208 * 208 // 8, _BF16)
    f1 = f1.reshape(n, 208, 208, 32)

    f2 = _cbl_s2(f1, p["b2"])
    feat_small = _cbl_s2(f2, p["b3"])
    feat_medium = _cbl_s2(feat_small, p["b4"])
    feat_large = _cbl_s2(feat_medium, p["b5"])

    def cbl1(xs, pp, out_dtype=_BF16):
        return _pw(xs, pp["w"], pp["scale"], pp["bias"], leaky=True,
                   out_dtype=out_dtype)

    s = cbl1([feat_large], p["spp_c1"])
    m5, m9, m13 = _spp_pools(s)
    p5 = cbl1([s, m5, m9, m13], p["spp_c2"])

    p5_up = _upsample2(cbl1([p5], p["conv_up1"]))
    p4 = cbl1([feat_medium], p["lateral1"])
    p4 = cbl1([p4, p5_up], p["merge1_0"])
    p4 = _cbl_s1_3x3(p4, p["merge1_1"])
    p4 = cbl1([p4], p["merge1_2"])

    p4_up = _upsample2(cbl1([p4], p["conv_up2"]))
    p3 = cbl1([feat_small], p["lateral2"])
    p3 = cbl1([p3, p4_up], p["merge2_0"])
    p3 = _cbl_s1_3x3(p3, p["merge2_1"])
    p3 = cbl1([p3], p["merge2_2"])

    def head(feat, p0, p1):
        hh = _cbl_s1_3x3(feat, p0)
        c_out = p1["w"].shape[0]
        out = _pw([hh], p1["w"], jnp.ones((c_out,), _F32), p1["b"],
                  leaky=False, out_dtype=_F32)
        return jnp.transpose(out, (0, 3, 1, 2))

    out_small = head(p3, p["head_s_0"], p["head_s_1"])
    out_medium = head(p4, p["head_m_0"], p["head_m_1"])
    out_large = head(p5, p["head_l_0"], p["head_l_1"])
    return (out_small, out_medium, out_large)


# speedup vs baseline: 8.4857x; 3.5964x over previous
import functools

import jax
import jax.numpy as jnp
import numpy as np
from jax.experimental import pallas as pl

_F32 = jnp.float32
_BF16 = jnp.bfloat16


def _epilogue(acc, s_ref, b_ref, leaky, out_dtype):
    y = acc * s_ref[...] + b_ref[...]
    if leaky:
        y = jnp.where(y > 0, y, _F32(0.1) * y)
    return y.astype(out_dtype)


def _call_conv(body, xs, wtap, scale, bias, m_rows, out_dtype):
    g_num = xs[0].shape[0]
    k_dim, c_out = wtap.shape
    return pl.pallas_call(
        body,
        grid=(g_num,),
        in_specs=[pl.BlockSpec((1,) + x.shape[1:], lambda g: (g, 0, 0))
                  for x in xs] + [
            pl.BlockSpec((k_dim, c_out), lambda g: (0, 0)),
            pl.BlockSpec((1, c_out), lambda g: (0, 0)),
            pl.BlockSpec((1, c_out), lambda g: (0, 0)),
        ],
        out_specs=pl.BlockSpec((1, m_rows, c_out), lambda g: (g, 0, 0)),
        out_shape=jax.ShapeDtypeStruct((g_num, m_rows, c_out), out_dtype),
    )(*[x.astype(_BF16) for x in xs], wtap.astype(_BF16),
      scale.reshape(1, c_out).astype(_F32), bias.reshape(1, c_out).astype(_F32))


def _pw_body(*refs, n_in, leaky, out_dtype):
    x_refs, w_ref, s_ref, b_ref, o_ref = refs[:n_in], refs[n_in], refs[n_in + 1], refs[n_in + 2], refs[n_in + 3]
    if n_in == 1:
        x = x_refs[0][0]
    else:
        x = jnp.concatenate([r[0] for r in x_refs], axis=-1)
    acc = jnp.dot(x, w_ref[...], preferred_element_type=_F32)
    o_ref[0] = _epilogue(acc, s_ref, b_ref, leaky, out_dtype)


def _pw(xs, w_oi11, scale, bias, *, leaky, out_dtype=_BF16):
    n, h, w, _ = xs[0].shape
    c_out, c_in = w_oi11.shape[0], w_oi11.shape[1]
    wt = w_oi11[:, :, 0, 0].T
    body = functools.partial(_pw_body, n_in=len(xs), leaky=leaky,
                             out_dtype=out_dtype)
    out = _call_conv(body, [x.reshape(n, h * w, x.shape[3]) for x in xs],
                     wt, scale, bias, h * w, out_dtype)
    return out.reshape(n, h, w, c_out)


def _s1_body(x_ref, w_ref, s_ref, b_ref, o_ref, *, h, w, c, leaky, out_dtype):
    m = h * w
    a = x_ref[0]
    z = jnp.zeros((w + 1, c), _BF16)
    ax = jnp.concatenate([z, a, z], axis=0)
    col = jax.lax.broadcasted_iota(jnp.int32, (m, 1), 0) % w
    first, last = col == 0, col == (w - 1)
    parts = []
    for dy in range(3):
        for dx in range(3):
            s = jax.lax.slice(ax, (dy * w + dx, 0), (dy * w + dx + m, c))
            if dx == 0:
                s = jnp.where(first, _BF16(0), s)
            elif dx == 2:
                s = jnp.where(last, _BF16(0), s)
            parts.append(s)
    x = jnp.concatenate(parts, axis=-1)
    acc = jnp.dot(x, w_ref[...], preferred_element_type=_F32)
    o_ref[0] = _epilogue(acc, s_ref, b_ref, leaky, out_dtype)


def _cbl_s1_3x3(x, p):
    n, h, w, c = x.shape
    wt = jnp.transpose(p["w"], (2, 3, 1, 0)).reshape(9 * c, -1)
    body = functools.partial(_s1_body, h=h, w=w, c=c, leaky=True,
                             out_dtype=_BF16)
    out = _call_conv(body, [x.reshape(n, h * w, c)], wt, p["scale"], p["bias"],
                     h * w, _BF16)
    return out.reshape(n, h, w, -1)


def _s2_body(x_ref, w_ref, s_ref, b_ref, o_ref, *, h, w, c, leaky, out_dtype):
    h2, w2 = h // 2, w // 2
    m = h2 * w2
    a = x_ref[0].reshape(h2, 2, w2, 2, c)
    planes = [a[:, ry, :, rx, :].reshape(m, c)
              for ry in (0, 1) for rx in (0, 1)]
    p4 = jnp.concatenate(planes, axis=-1)
    ext = jnp.concatenate([p4, jnp.zeros((w2 + 1, 4 * c), _BF16)], axis=0)
    col = jax.lax.broadcasted_iota(jnp.int32, (m, 1), 0) % w2
    last = col == (w2 - 1)
    parts = []
    for qy in (0, 1):
        for qx in (0, 1):
            s = jax.lax.slice(ext, (qy * w2 + qx, 0), (qy * w2 + qx + m, 4 * c))
            if qx == 1:
                s = jnp.where(last, _BF16(0), s)
            parts.append(s)
    x = jnp.concatenate(parts, axis=-1)
    acc = jnp.dot(x, w_ref[...], preferred_element_type=_F32)
    o_ref[0] = _epilogue(acc, s_ref, b_ref, leaky, out_dtype)


def _cbl_s2(x, p):
    n, h, w, c = x.shape
    o = p["w"].shape[0]
    wp = jnp.pad(p["w"], ((0, 0), (0, 0), (0, 1), (0, 1)))
    wr = wp.reshape(o, c, 2, 2, 2, 2)
    wt = jnp.transpose(wr, (2, 4, 3, 5, 1, 0)).reshape(16 * c, o)
    body = functools.partial(_s2_body, h=h, w=w, c=c, leaky=True,
                             out_dtype=_BF16)
    out = _call_conv(body, [x.reshape(n, h * w, c)], wt, p["scale"], p["bias"],
                     (h // 2) * (w // 2), _BF16)
    return out.reshape(n, h // 2, w // 2, o)


def _pool_body(x_ref, o5_ref, o9_ref, o13_ref):
    a = x_ref[0]

    def pool5(v):
        h2, w2 = v.shape[0] - 4, v.shape[1] - 4
        r = v[0:h2]
        for i in range(1, 5):
            r = jnp.maximum(r, v[i:i + h2])
        cc = r[:, 0:w2]
        for i in range(1, 5):
            cc = jnp.maximum(cc, r[:, i:i + w2])
        return cc

    m5 = pool5(a)
    m9 = pool5(m5)
    m13 = pool5(m9)
    o5_ref[0] = m5[4:17, 4:17]
    o9_ref[0] = m9[2:15, 2:15]
    o13_ref[0] = m13


def _spp_pools(s):
    n, h, w, c = s.shape
    pad = jnp.full((n, h + 12, w + 12, c), _BF16(-1e30))
    canvas = pad.at[:, 6:6 + h, 6:6 + w, :].set(s)
    shp = jax.ShapeDtypeStruct((n, h, w, c), _BF16)
    return pl.pallas_call(
        _pool_body,
        grid=(n,),
        in_specs=[pl.BlockSpec((1, h + 12, w + 12, c), lambda g: (g, 0, 0, 0))],
        out_specs=[pl.BlockSpec((1, h, w, c), lambda g: (g, 0, 0, 0))] * 3,
        out_shape=[shp, shp, shp],
    )(canvas)


def _upsample2(x):
    n, h, w, c = x.shape
    x = jnp.broadcast_to(x[:, :, None, :, None, :], (n, h, 2, w, 2, c))
    return x.reshape(n, 2 * h, 2 * w, c)


_B1_TILES = 4
_B1_OX = 208 // _B1_TILES
_B1_XL = 2 * _B1_OX
_B1_K = 3 * _B1_XL
_B1_N = 32 * _B1_OX


def _b1_bands(w):
    oxl = np.arange(_B1_OX)
    dxv = np.arange(3)
    cv = np.arange(3)
    ov = np.arange(32)
    c4, x4, d4, o4 = np.meshgrid(cv, oxl, dxv, ov, indexing="ij")
    rows = (c4 * _B1_XL + 2 * x4 + d4).reshape(-1)
    cols = (32 * x4 + o4).reshape(-1)
    valid = ((2 * x4 + d4) < _B1_XL).reshape(-1)
    rows_v, cols_v = rows[valid], cols[valid]
    bands = []
    for qy, ry in ((0, 0), (0, 1), (1, 0)):
        dy = 2 * qy + ry
        wt = jnp.transpose(w[:, :, dy, :], (1, 2, 0))
        vals = jnp.broadcast_to(wt[:, None, :, :], (3, _B1_OX, 3, 32))
        vals_v = vals.reshape(-1)[valid]
        bm = jnp.zeros((_B1_K, _B1_N), _F32).at[rows_v, cols_v].set(vals_v)
        be = jnp.zeros((_B1_K, _B1_N), _F32).at[
            np.repeat(cv * _B1_XL, 32),
            np.tile(32 * (_B1_OX - 1) + ov, 3)].set(wt[:, 2, :].reshape(-1))
        bands.append(jnp.stack([bm, be], axis=0))
    return jnp.stack(bands, axis=0).astype(_BF16)


def _b1_body(x_ref, w_ref, s_ref, b_ref, o_ref):
    a = x_ref[0].astype(_BF16)
    ap = a.reshape(3, 208, 2, 416)
    z = jnp.zeros((1, _B1_K), _BF16)

    def lhs(t, qy, ry):
        parts = [ap[c, :, ry, t * _B1_XL:(t + 1) * _B1_XL] for c in range(3)]
        l = jnp.concatenate(parts, axis=-1)
        if qy == 1:
            l = jnp.concatenate([l[1:], z], axis=0)
        return l

    for t in range(_B1_TILES):
        acc = None
        for i, (qy, ry) in enumerate(((0, 0), (0, 1), (1, 0))):
            s = jnp.dot(lhs(t, qy, ry), w_ref[i, 0],
                        preferred_element_type=_F32)
            if t + 1 < _B1_TILES:
                s = s + jnp.dot(lhs(t + 1, qy, ry), w_ref[i, 1],
                                preferred_element_type=_F32)
            acc = s if acc is None else acc + s
        o_ref[0, :, t * _B1_N:(t + 1) * _B1_N] = _epilogue(
            acc, s_ref, b_ref, True, _BF16)


def _b1_conv(x, p):
    n = x.shape[0]
    bands = _b1_bands(p["w"].astype(_F32))
    s_t = jnp.tile(p["scale"], _B1_OX).reshape(1, _B1_N)
    b_t = jnp.tile(p["bias"], _B1_OX).reshape(1, _B1_N)
    out = pl.pallas_call(
        _b1_body,
        grid=(n,),
        in_specs=[
            pl.BlockSpec((1, 3, 416, 416), lambda i: (i, 0, 0, 0)),
            pl.BlockSpec((3, 2, _B1_K, _B1_N), lambda i: (0, 0, 0, 0)),
            pl.BlockSpec((1, _B1_N), lambda i: (0, 0)),
            pl.BlockSpec((1, _B1_N), lambda i: (0, 0)),
        ],
        out_specs=pl.BlockSpec((1, 208, 208 * 32), lambda i: (i, 0, 0)),
        out_shape=jax.ShapeDtypeStruct((n, 208, 208 * 32), _BF16),
    )(x, bands, s_t, b_t)
    return out.reshape(n, 208, 208, 32)


def kernel(x, params):
    p = params
    n = x.shape[0]

    f1 = _b1_conv(x, p["b1"])

    f2 = _cbl_s2(f1, p["b2"])
    feat_small = _cbl_s2(f2, p["b3"])
    feat_medium = _cbl_s2(feat_small, p["b4"])
    feat_large = _cbl_s2(feat_medium, p["b5"])

    def cbl1(xs, pp, out_dtype=_BF16):
        return _pw(xs, pp["w"], pp["scale"], pp["bias"], leaky=True,
                   out_dtype=out_dtype)

    s = cbl1([feat_large], p["spp_c1"])
    m5, m9, m13 = _spp_pools(s)
    p5 = cbl1([s, m5, m9, m13], p["spp_c2"])

    p5_up = _upsample2(cbl1([p5], p["conv_up1"]))
    p4 = cbl1([feat_medium], p["lateral1"])
    p4 = cbl1([p4, p5_up], p["merge1_0"])
    p4 = _cbl_s1_3x3(p4, p["merge1_1"])
    p4 = cbl1([p4], p["merge1_2"])

    p4_up = _upsample2(cbl1([p4], p["conv_up2"]))
    p3 = cbl1([feat_small], p["lateral2"])
    p3 = cbl1([p3, p4_up], p["merge2_0"])
    p3 = _cbl_s1_3x3(p3, p["merge2_1"])
    p3 = cbl1([p3], p["merge2_2"])

    def head(feat, p0, p1):
        hh = _cbl_s1_3x3(feat, p0)
        c_out = p1["w"].shape[0]
        out = _pw([hh], p1["w"], jnp.ones((c_out,), _F32), p1["b"],
                  leaky=False, out_dtype=_F32)
        return jnp.transpose(out, (0, 3, 1, 2))

    out_small = head(p3, p["head_s_0"], p["head_s_1"])
    out_medium = head(p4, p["head_m_0"], p["head_m_1"])
    out_large = head(p5, p["head_l_0"], p["head_l_1"])
    return (out_small, out_medium, out_large)
